# baseline (speedup 1.0000x reference)
import functools
import math

import jax
import jax.numpy as jnp
from jax import lax
from jax.experimental import pallas as pl
from jax.experimental.pallas import tpu as pltpu
from jax.experimental.pallas import tpu_sc as plsc

_F = 6
_E = 64
_NCH = 16
_EC = 2
_B_TC = 5


def _pe_face_kernel(ftT_ref, w1T_ref, b1_ref, w2T_ref, b2_ref, pe_ref,
                    scratch, sems, *, H, W):
    CH = H // _NCH
    CW = CH * W
    w1T = w1T_ref[...]
    ftT = ftT_ref[...]

    def copies(c, f):
        src = scratch.at[f, :, pl.ds(c * CH, CH), :]
        return [pltpu.make_async_copy(
            src, pe_ref.at[f, :, pl.ds(c * CH, CH), :], sems.at[f])]

    for c in range(_NCH):
        j = lax.broadcasted_iota(jnp.int32, (1, CW), 1) + c * CW
        x_row = (j % W).astype(jnp.float32) * (2.0 / (W - 1)) - 1.0
        y_row = (j // W).astype(jnp.float32) * (2.0 / (H - 1)) - 1.0
        hT = w1T[:, 0:1] * x_row + w1T[:, 1:2] * y_row + b1_ref[...]
        hT = hT * 0.5 * (1.0 + lax.erf(hT * (1.0 / math.sqrt(2.0))))
        ceT = jax.lax.dot_general(
            w2T_ref[...], hT, (((1,), (0,)), ((), ())),
            preferred_element_type=jnp.float32,
            precision=lax.Precision.HIGHEST) + b2_ref[...]
        for f in range(_F):
            scratch[f, :, c * CH:(c + 1) * CH, :] = (
                ceT + ftT[:, f:f + 1]).reshape(_E, CH, W)
            for cp in copies(c, f):
                cp.start()
    for c in range(_NCH):
        for f in range(_F):
            for cp in copies(c, f):
                cp.wait()


def _compute_pe_face(face_table, W1, b1, W2, b2, H, W):
    return pl.pallas_call(
        functools.partial(_pe_face_kernel, H=H, W=W),
        in_specs=[pl.BlockSpec(memory_space=pltpu.VMEM)] * 5,
        out_specs=pl.BlockSpec(memory_space=pltpu.HBM),
        out_shape=jax.ShapeDtypeStruct((_F, _E, H, W), jnp.float32),
        scratch_shapes=[
            pltpu.VMEM((_F, _E, H, W), jnp.float32),
            pltpu.SemaphoreType.DMA((_F,)),
        ],
    )(face_table.T, W1.T, b1[:, None], W2.T, b2[:, None])


def _tc_bcast_kernel(pe_ref, out_ref, sems, *, n_tc):
    cps = [pltpu.make_async_copy(
        pe_ref.at[f], out_ref.at[b * _F + f], sems.at[b * _F + f])
        for b in range(n_tc) for f in range(_F)]
    for cp in cps:
        cp.start()
    for cp in cps:
        cp.wait()


def _tc_broadcast(pe_face, n_tc, H, W):
    return pl.pallas_call(
        functools.partial(_tc_bcast_kernel, n_tc=n_tc),
        in_specs=[pl.BlockSpec(memory_space=pltpu.HBM)],
        out_specs=pl.BlockSpec(memory_space=pltpu.HBM),
        out_shape=jax.ShapeDtypeStruct((n_tc * _F, _E, H, W), jnp.float32),
        scratch_shapes=[pltpu.SemaphoreType.DMA((n_tc * _F,))],
    )(pe_face)


def _sc_broadcast(pe_face, n_sc, H, W):
    n_items = _F * (_E // _EC)
    n_workers = 32
    per_w = n_items // n_workers
    mesh = plsc.VectorSubcoreMesh(core_axis_name="c", subcore_axis_name="s")

    @functools.partial(
        pl.kernel, mesh=mesh,
        out_type=jax.ShapeDtypeStruct((n_sc * _F, _E, H, W), jnp.float32),
        scratch_types=[
            pltpu.VMEM((_EC, H, W), jnp.float32),
            pltpu.VMEM((_EC, H, W), jnp.float32),
            pltpu.SemaphoreType.DMA((2,)),
        ],
    )
    def bcast(pe_ref, out_ref, buf0, buf1, sems):
        wid = lax.axis_index("s") * 2 + lax.axis_index("c")
        bufs = (buf0, buf1)

        def item_copies(it, buf, slot):
            f = it // (_E // _EC)
            e0 = (it % (_E // _EC)) * _EC
            read = pltpu.make_async_copy(
                pe_ref.at[f, pl.ds(e0, _EC)], buf, sems.at[slot])
            writes = [pltpu.make_async_copy(
                buf, out_ref.at[b * _F + f, pl.ds(e0, _EC)], sems.at[slot])
                for b in range(n_sc)]
            return read, writes

        prev = [None, None]
        for i in range(per_w):
            slot = i % 2
            it = wid * per_w + i
            read, writes = item_copies(it, bufs[slot], slot)
            if prev[slot] is not None:
                for cp in prev[slot]:
                    cp.wait()
            read.start()
            read.wait()
            for cp in writes:
                cp.start()
            prev[slot] = writes
        for slot in range(2):
            if prev[slot] is not None:
                for cp in prev[slot]:
                    cp.wait()

    return bcast(pe_face)


def kernel(latents, face_table, W1, b1, W2, b2):
    BF, _C, H, W = latents.shape
    B = BF // _F
    n_tc = min(_B_TC, B)
    n_sc = B - n_tc
    pe_face = _compute_pe_face(face_table, W1, b1, W2, b2, H, W)
    out_sc = _sc_broadcast(pe_face, n_sc, H, W)
    out_tc = _tc_broadcast(pe_face, n_tc, H, W)
    return jnp.concatenate([out_tc, out_sc], axis=0)


# speedup vs baseline: 16.4695x; 16.4695x over previous
import functools
import math

import jax
import jax.numpy as jnp
from jax import lax
from jax.experimental import pallas as pl
from jax.experimental.pallas import tpu as pltpu
from jax.experimental.pallas import tpu_sc as plsc

_F = 6
_E = 64
_NCH = 16
_EC = 2
_B_TC = 5


def _pe_face_kernel(ftT_ref, w1T_ref, b1_ref, w2T_ref, b2_ref, pe_ref,
                    scratch, sems, *, H, W):
    CH = H // _NCH
    CW = CH * W
    w1T = w1T_ref[...]
    ftT = ftT_ref[...]

    def copies(c, f):
        src = scratch.at[f, :, pl.ds(c * CH, CH), :]
        return [pltpu.make_async_copy(
            src, pe_ref.at[f, :, pl.ds(c * CH, CH), :], sems.at[f])]

    for c in range(_NCH):
        j = lax.broadcasted_iota(jnp.int32, (1, CW), 1) + c * CW
        x_row = (j % W).astype(jnp.float32) * (2.0 / (W - 1)) - 1.0
        y_row = (j // W).astype(jnp.float32) * (2.0 / (H - 1)) - 1.0
        hT = w1T[:, 0:1] * x_row + w1T[:, 1:2] * y_row + b1_ref[...]
        hT = hT * 0.5 * (1.0 + lax.erf(hT * (1.0 / math.sqrt(2.0))))
        ceT = jax.lax.dot_general(
            w2T_ref[...], hT, (((1,), (0,)), ((), ())),
            preferred_element_type=jnp.float32,
            precision=lax.Precision.HIGHEST) + b2_ref[...]
        for f in range(_F):
            scratch[f, :, c * CH:(c + 1) * CH, :] = (
                ceT + ftT[:, f:f + 1]).reshape(_E, CH, W)
            for cp in copies(c, f):
                cp.start()
    for c in range(_NCH):
        for f in range(_F):
            for cp in copies(c, f):
                cp.wait()


def _compute_pe_face(face_table, W1, b1, W2, b2, H, W):
    return pl.pallas_call(
        functools.partial(_pe_face_kernel, H=H, W=W),
        in_specs=[pl.BlockSpec(memory_space=pltpu.VMEM)] * 5,
        out_specs=pl.BlockSpec(memory_space=pltpu.HBM),
        out_shape=jax.ShapeDtypeStruct((_F, _E, H, W), jnp.float32),
        scratch_shapes=[
            pltpu.VMEM((_F, _E, H, W), jnp.float32),
            pltpu.SemaphoreType.DMA((_F,)),
        ],
    )(face_table.T, W1.T, b1[:, None], W2.T, b2[:, None])


def _tc_bcast_kernel(pe_ref, out_ref, scratch, rsems, wsems, *, n_tc):
    reads = [pltpu.make_async_copy(pe_ref.at[f], scratch.at[f], rsems.at[f])
             for f in range(_F)]
    for cp in reads:
        cp.start()
    writes = []
    for f in range(_F):
        reads[f].wait()
        for b in range(n_tc):
            cp = pltpu.make_async_copy(
                scratch.at[f], out_ref.at[b * _F + f], wsems.at[b * _F + f])
            cp.start()
            writes.append(cp)
    for cp in writes:
        cp.wait()


def _tc_broadcast(pe_face, n_tc, H, W):
    return pl.pallas_call(
        functools.partial(_tc_bcast_kernel, n_tc=n_tc),
        in_specs=[pl.BlockSpec(memory_space=pltpu.HBM)],
        out_specs=pl.BlockSpec(memory_space=pltpu.HBM),
        out_shape=jax.ShapeDtypeStruct((n_tc * _F, _E, H, W), jnp.float32),
        scratch_shapes=[
            pltpu.VMEM((_F, _E, H, W), jnp.float32),
            pltpu.SemaphoreType.DMA((_F,)),
            pltpu.SemaphoreType.DMA((n_tc * _F,)),
        ],
    )(pe_face)


def _sc_broadcast(pe_face, n_sc, H, W):
    n_items = _F * (_E // _EC)
    n_workers = 32
    per_w = n_items // n_workers
    mesh = plsc.VectorSubcoreMesh(core_axis_name="c", subcore_axis_name="s")

    @functools.partial(
        pl.kernel, mesh=mesh,
        out_type=jax.ShapeDtypeStruct((n_sc * _F, _E, H, W), jnp.float32),
        scratch_types=[
            pltpu.VMEM((_EC, H, W), jnp.float32),
            pltpu.VMEM((_EC, H, W), jnp.float32),
            pltpu.SemaphoreType.DMA((2,)),
        ],
    )
    def bcast(pe_ref, out_ref, buf0, buf1, sems):
        wid = lax.axis_index("s") * 2 + lax.axis_index("c")
        bufs = (buf0, buf1)

        def item_copies(it, buf, slot):
            f = it // (_E // _EC)
            e0 = (it % (_E // _EC)) * _EC
            read = pltpu.make_async_copy(
                pe_ref.at[f, pl.ds(e0, _EC)], buf, sems.at[slot])
            writes = [pltpu.make_async_copy(
                buf, out_ref.at[b * _F + f, pl.ds(e0, _EC)], sems.at[slot])
                for b in range(n_sc)]
            return read, writes

        prev = [None, None]
        for i in range(per_w):
            slot = i % 2
            it = wid * per_w + i
            read, writes = item_copies(it, bufs[slot], slot)
            if prev[slot] is not None:
                for cp in prev[slot]:
                    cp.wait()
            read.start()
            read.wait()
            for cp in writes:
                cp.start()
            prev[slot] = writes
        for slot in range(2):
            if prev[slot] is not None:
                for cp in prev[slot]:
                    cp.wait()

    return bcast(pe_face)


def kernel(latents, face_table, W1, b1, W2, b2):
    BF, _C, H, W = latents.shape
    B = BF // _F
    n_tc = min(_B_TC, B)
    n_sc = B - n_tc
    pe_face = _compute_pe_face(face_table, W1, b1, W2, b2, H, W)
    out_sc = _sc_broadcast(pe_face, n_sc, H, W)
    out_tc = _tc_broadcast(pe_face, n_tc, H, W)
    return jnp.concatenate([out_tc, out_sc], axis=0)


# speedup vs baseline: 35.6334x; 2.1636x over previous
import functools
import math

import jax
import jax.numpy as jnp
from jax import lax
from jax.experimental import pallas as pl
from jax.experimental.pallas import tpu as pltpu
from jax.experimental.pallas import tpu_sc as plsc

_F = 6
_E = 64
_NCH = 16
_EC = 2
_NBUF = 3


def _pe_face_kernel(ftT_ref, w1T_ref, b1_ref, w2T_ref, b2_ref, pe_ref,
                    scratch, sems, *, H, W):
    CH = H // _NCH
    CW = CH * W
    w1T = w1T_ref[...]
    ftT = ftT_ref[...]

    def copy(c, f):
        return pltpu.make_async_copy(
            scratch.at[f, :, pl.ds(c * CH, CH), :],
            pe_ref.at[f, :, pl.ds(c * CH, CH), :], sems.at[f])

    for c in range(_NCH):
        j = lax.broadcasted_iota(jnp.int32, (1, CW), 1) + c * CW
        x_row = (j % W).astype(jnp.float32) * (2.0 / (W - 1)) - 1.0
        y_row = (j // W).astype(jnp.float32) * (2.0 / (H - 1)) - 1.0
        hT = w1T[:, 0:1] * x_row + w1T[:, 1:2] * y_row + b1_ref[...]
        hT = hT * 0.5 * (1.0 + lax.erf(hT * (1.0 / math.sqrt(2.0))))
        ceT = jax.lax.dot_general(
            w2T_ref[...], hT, (((1,), (0,)), ((), ())),
            preferred_element_type=jnp.float32,
            precision=lax.Precision.HIGHEST) + b2_ref[...]
        for f in range(_F):
            scratch[f, :, c * CH:(c + 1) * CH, :] = (
                ceT + ftT[:, f:f + 1]).reshape(_E, CH, W)
            copy(c, f).start()
    for c in range(_NCH):
        for f in range(_F):
            copy(c, f).wait()


def _compute_pe_face(face_table, W1, b1, W2, b2, H, W):
    return pl.pallas_call(
        functools.partial(_pe_face_kernel, H=H, W=W),
        in_specs=[pl.BlockSpec(memory_space=pltpu.VMEM)] * 5,
        out_specs=pl.BlockSpec(memory_space=pltpu.HBM),
        out_shape=jax.ShapeDtypeStruct((_F, _E, H, W), jnp.float32),
        scratch_shapes=[
            pltpu.VMEM((_F, _E, H, W), jnp.float32),
            pltpu.SemaphoreType.DMA((_F,)),
        ],
    )(face_table.T, W1.T, b1[:, None], W2.T, b2[:, None])


def _sc_broadcast(pe_face, B, H, W):
    n_items = _F * (_E // _EC)
    n_workers = 32
    per_w = n_items // n_workers
    mesh = plsc.VectorSubcoreMesh(core_axis_name="c", subcore_axis_name="s")

    @functools.partial(
        pl.kernel, mesh=mesh,
        out_type=jax.ShapeDtypeStruct((B * _F, _E, H, W), jnp.float32),
        scratch_types=[
            [pltpu.VMEM((_EC, H, W), jnp.float32) for _ in range(_NBUF)],
            pltpu.SemaphoreType.DMA((_NBUF,)),
            pltpu.SemaphoreType.DMA((_NBUF,)),
        ],
    )
    def bcast(pe_ref, out_ref, bufs, rsems, wsems):
        wid = lax.axis_index("s") * 2 + lax.axis_index("c")

        def fe(i):
            it = wid * per_w + i
            return it // (_E // _EC), (it % (_E // _EC)) * _EC

        def read(i):
            f, e0 = fe(i)
            return pltpu.make_async_copy(
                pe_ref.at[f, pl.ds(e0, _EC)], bufs[i % _NBUF],
                rsems.at[i % _NBUF])

        def writes(i):
            f, e0 = fe(i)
            return [pltpu.make_async_copy(
                bufs[i % _NBUF], out_ref.at[b * _F + f, pl.ds(e0, _EC)],
                wsems.at[i % _NBUF]) for b in range(B)]

        pending = [None] * _NBUF
        read(0).start()
        for i in range(per_w):
            slot = i % _NBUF
            read(i).wait()
            if i + 1 < per_w:
                nslot = (i + 1) % _NBUF
                if pending[nslot] is not None:
                    for cp in pending[nslot]:
                        cp.wait()
                    pending[nslot] = None
                read(i + 1).start()
            ws = writes(i)
            for cp in ws:
                cp.start()
            pending[slot] = ws
        for slot in range(_NBUF):
            if pending[slot] is not None:
                for cp in pending[slot]:
                    cp.wait()

    return bcast(pe_face)


def kernel(latents, face_table, W1, b1, W2, b2):
    BF, _C, H, W = latents.shape
    B = BF // _F
    pe_face = _compute_pe_face(face_table, W1, b1, W2, b2, H, W)
    return _sc_broadcast(pe_face, B, H, W)
